# Initial kernel scaffold; baseline (speedup 1.0000x reference)
import jax
import jax.numpy as jnp
from jax.experimental import pallas as pl


def kernel(x, edge_index, edge_attr, batch, params):
    raise NotImplementedError("write your pallas kernel here")


# speedup vs baseline: 12.1085x; 12.1085x over previous
import functools
import numpy as np
import jax
import jax.numpy as jnp
from jax.experimental import pallas as pl
from jax.experimental.pallas import tpu as pltpu

N_NODES = 10000
N_EDGES = 160000
HID = 128
T = 4
F = 128
FO = 32
TF = T * F
NG = 64
AVG_LOG_C = float(np.log(np.arange(51, dtype=np.float64) + 1.0).sum() / 51.0)
EPS_BN = 1e-5


def _mm_body(x_ref, w_ref, b_ref, o_ref):
    o_ref[...] = (jnp.dot(x_ref[...], w_ref[...], preferred_element_type=jnp.float32)
                  + b_ref[...])


def _mm(x, w, b, block_m):
    M, K = x.shape
    Ko = w.shape[1]
    return pl.pallas_call(
        _mm_body,
        grid=(M // block_m,),
        in_specs=[pl.BlockSpec((block_m, K), lambda i: (i, 0)),
                  pl.BlockSpec((K, Ko), lambda i: (0, 0)),
                  pl.BlockSpec((1, Ko), lambda i: (0, 0))],
        out_specs=pl.BlockSpec((block_m, Ko), lambda i: (i, 0)),
        out_shape=jax.ShapeDtypeStruct((M, Ko), jnp.float32),
    )(x, w, b.reshape(1, -1))


def _fold_body(ew_ref, ebr_ref, cew_ref, ceb_ref, we_ref, mc_ref, cc_ref):
    t1 = jnp.dot(ew_ref[...], cew_ref[0], preferred_element_type=jnp.float32)
    mc_ref[0] = jnp.dot(t1, we_ref[0], preferred_element_type=jnp.float32)
    t2 = jnp.dot(ebr_ref[...], cew_ref[0], preferred_element_type=jnp.float32) + ceb_ref[0]
    cc_ref[0] = jnp.dot(t2, we_ref[0], preferred_element_type=jnp.float32)


def _fold_edge_weights(eW, eb, ceW_all, ceb_all, WE_all):
    L = ceW_all.shape[0]
    return pl.pallas_call(
        _fold_body,
        grid=(L,),
        in_specs=[pl.BlockSpec((4, HID), lambda i: (0, 0)),
                  pl.BlockSpec((1, HID), lambda i: (0, 0)),
                  pl.BlockSpec((1, HID, HID), lambda i: (i, 0, 0)),
                  pl.BlockSpec((1, 1, HID), lambda i: (i, 0, 0)),
                  pl.BlockSpec((1, HID, TF), lambda i: (i, 0, 0))],
        out_specs=[pl.BlockSpec((1, 4, TF), lambda i: (i, 0, 0)),
                   pl.BlockSpec((1, 1, TF), lambda i: (i, 0, 0))],
        out_shape=[jax.ShapeDtypeStruct((L, 4, TF), jnp.float32),
                   jax.ShapeDtypeStruct((L, 1, TF), jnp.float32)],
    )(eW, eb.reshape(1, -1), ceW_all, ceb_all.reshape(L, 1, HID), WE_all)


def _post_body(a_ref, s1_ref, s2_ref, mx_ref, mn_ref, h_ref, cnt_ref,
               whf_ref, wst_ref, pb_ref, lw_ref, lb_ref,
               bg_ref, bb_ref, brm_ref, brv_ref, o_ref):
    cnt = cnt_ref[...]
    cntc = jnp.maximum(cnt, 1.0)
    has = cnt > 0.0
    inv = 1.0 / cntc
    logdeg = jnp.log(cntc + 1.0)
    ld = logdeg * (1.0 / AVG_LOG_C)
    li = AVG_LOG_C / logdeg

    A = a_ref[...]
    Ed = s1_ref[...] * inv
    Ed2 = s2_ref[...] * inv
    mean = jnp.where(has, A + Ed, 0.0)
    mx = jnp.where(has, A + mx_ref[...], 0.0)
    mn = jnp.where(has, A + mn_ref[...], 0.0)
    var = jnp.where(has, jnp.maximum(Ed2 - Ed * Ed, 0.0), 0.0)
    std = jnp.sqrt(var + 1e-5)

    h = h_ref[...]
    out = jnp.dot(h, whf_ref[...], preferred_element_type=jnp.float32) + pb_ref[...]
    cols = []
    for t in range(T):
        sl = slice(t * F, (t + 1) * F)
        agg_t = jnp.concatenate([mean[:, sl], mx[:, sl], mn[:, sl], std[:, sl]], axis=1)
        Pt = jnp.dot(agg_t, wst_ref[t], preferred_element_type=jnp.float32)
        cols.append(Pt[:, :FO] + ld[:, :1] * Pt[:, FO:2 * FO] + li[:, :1] * Pt[:, 2 * FO:])
    out = out + jnp.concatenate(cols, axis=1)
    res = jnp.dot(out, lw_ref[...], preferred_element_type=jnp.float32) + lb_ref[...]
    k = bg_ref[...] * jax.lax.rsqrt(brv_ref[...] + EPS_BN)
    res = res * k + (bb_ref[...] - brm_ref[...] * k)
    o_ref[...] = jnp.maximum(res, 0.0) + h


def _post(A, S1, S2, MX, MN, h, cntf, Whf, Wstack, postb_f, linW, linb,
          bn_g, bn_b, bn_rm, bn_rv, block_m=1000):
    M = A.shape[0]
    r1 = lambda v: v.reshape(1, -1)
    big = lambda i: (i, 0)
    cst = lambda i: (0, 0)
    return pl.pallas_call(
        _post_body,
        grid=(M // block_m,),
        in_specs=[pl.BlockSpec((block_m, TF), big),
                  pl.BlockSpec((block_m, TF), big),
                  pl.BlockSpec((block_m, TF), big),
                  pl.BlockSpec((block_m, TF), big),
                  pl.BlockSpec((block_m, TF), big),
                  pl.BlockSpec((block_m, HID), big),
                  pl.BlockSpec((block_m, 1), big),
                  pl.BlockSpec((HID, HID), cst),
                  pl.BlockSpec((T, TF, 3 * FO), lambda i: (0, 0, 0)),
                  pl.BlockSpec((1, HID), cst),
                  pl.BlockSpec((HID, HID), cst),
                  pl.BlockSpec((1, HID), cst),
                  pl.BlockSpec((1, HID), cst),
                  pl.BlockSpec((1, HID), cst),
                  pl.BlockSpec((1, HID), cst),
                  pl.BlockSpec((1, HID), cst)],
        out_specs=pl.BlockSpec((block_m, HID), big),
        out_shape=jax.ShapeDtypeStruct((M, HID), jnp.float32),
    )(A, S1, S2, MX, MN, h, cntf.reshape(-1, 1), Whf, Wstack, r1(postb_f),
      linW, r1(linb), r1(bn_g), r1(bn_b), r1(bn_rm), r1(bn_rv))


def _pool_body(b_ref, h_ref, o_ref, c_ref):
    i = pl.program_id(0)
    bb = b_ref[0]
    oh = (jax.lax.broadcasted_iota(jnp.int32, (NG, bb.shape[1]), 0) == bb
          ).astype(jnp.float32)
    acc = jnp.dot(oh, h_ref[...], preferred_element_type=jnp.float32)
    cacc = jnp.sum(oh, axis=1, keepdims=True) * jnp.ones((1, HID), jnp.float32)

    @pl.when(i == 0)
    def _():
        o_ref[...] = acc
        c_ref[...] = cacc

    @pl.when(i > 0)
    def _():
        o_ref[...] += acc
        c_ref[...] += cacc


def _pool(batch, h, block_m=1000):
    M = h.shape[0]
    nb = M // block_m
    b3 = batch.reshape(nb, 1, block_m)
    return pl.pallas_call(
        _pool_body,
        grid=(nb,),
        in_specs=[pl.BlockSpec((1, 1, block_m), lambda i: (i, 0, 0)),
                  pl.BlockSpec((block_m, HID), lambda i: (i, 0))],
        out_specs=[pl.BlockSpec((NG, HID), lambda i: (0, 0)),
                   pl.BlockSpec((NG, HID), lambda i: (0, 0))],
        out_shape=[jax.ShapeDtypeStruct((NG, HID), jnp.float32),
                   jax.ShapeDtypeStruct((NG, HID), jnp.float32)],
    )(b3, h)


def _mlp_body(ap_ref, cr_ref, w1_ref, b1_ref, bg_ref, bb_ref, brm_ref, brv_ref,
              w2_ref, b2_ref, w3_ref, b3_ref, o_ref):
    addp = ap_ref[...]
    cr = jnp.maximum(cr_ref[...], 1.0)
    meanp = addp / cr
    g = jnp.concatenate([meanp, addp], axis=1)
    z = jnp.dot(g, w1_ref[...], preferred_element_type=jnp.float32) + b1_ref[...]
    k = bg_ref[...] * jax.lax.rsqrt(brv_ref[...] + EPS_BN)
    z = z * k + (bb_ref[...] - brm_ref[...] * k)
    z = jnp.maximum(z, 0.0)
    z = jnp.maximum(jnp.dot(z, w2_ref[...], preferred_element_type=jnp.float32)
                    + b2_ref[...], 0.0)
    o_ref[...] = jnp.dot(z, w3_ref[...], preferred_element_type=jnp.float32) + b3_ref[...]


def _mlp(addp, cntrep, clf):
    r1 = lambda v: v.reshape(1, -1)
    W3p = jnp.pad(clf['W3'], ((0, 0), (0, HID - 1)))
    b3p = jnp.pad(clf['b3'], (0, HID - 1))
    cst = lambda: pl.BlockSpec(None, None)
    specs = [pl.BlockSpec((NG, HID), lambda: (0, 0)),
             pl.BlockSpec((NG, HID), lambda: (0, 0)),
             pl.BlockSpec((2 * HID, HID), lambda: (0, 0)),
             pl.BlockSpec((1, HID), lambda: (0, 0)),
             pl.BlockSpec((1, HID), lambda: (0, 0)),
             pl.BlockSpec((1, HID), lambda: (0, 0)),
             pl.BlockSpec((1, HID), lambda: (0, 0)),
             pl.BlockSpec((1, HID), lambda: (0, 0)),
             pl.BlockSpec((HID, HID // 2), lambda: (0, 0)),
             pl.BlockSpec((1, HID // 2), lambda: (0, 0)),
             pl.BlockSpec((HID // 2, HID), lambda: (0, 0)),
             pl.BlockSpec((1, HID), lambda: (0, 0))]
    return pl.pallas_call(
        _mlp_body,
        grid=(),
        in_specs=specs,
        out_specs=pl.BlockSpec((NG, HID), lambda: (0, 0)),
        out_shape=jax.ShapeDtypeStruct((NG, HID), jnp.float32),
    )(addp, cntrep, clf['W1'], r1(clf['b1']), r1(clf['bn_g']), r1(clf['bn_b']),
      r1(clf['bn_rm']), r1(clf['bn_rv']), clf['W2'], r1(clf['b2']), W3p, r1(b3p))


def kernel(x, edge_index, edge_attr, batch, params):
    src, dst = edge_index[0], edge_index[1]

    perm = jnp.argsort(dst)
    dst_s = dst[perm]
    src_s = src[perm]
    ea_s = edge_attr[perm]
    row_ptr = jnp.searchsorted(
        dst_s, jnp.arange(N_NODES + 1, dtype=jnp.int32)).astype(jnp.int32)
    cntf = (row_ptr[1:] - row_ptr[:-1]).astype(jnp.float32)
    gptr = jnp.searchsorted(
        batch, jnp.arange(NG + 1, dtype=jnp.int32)).astype(jnp.int32)
    del gptr

    layers = params['layers']
    L = len(layers)

    WA_all, WB_all, WE_all = [], [], []
    for lp in layers:
        pw = lp['preW']
        WA_all.append(pw[:, :F, :].transpose(1, 0, 2).reshape(F, TF))
        WB_all.append(pw[:, F:2 * F, :].transpose(1, 0, 2).reshape(F, TF))
        WE_all.append(pw[:, 2 * F:, :].transpose(1, 0, 2).reshape(F, TF))
    WE_stack = jnp.stack(WE_all)
    ceW_all = jnp.stack([lp['ceW'] for lp in layers])
    ceb_all = jnp.stack([lp['ceb'] for lp in layers])

    MC_all, cC_all = _fold_edge_weights(params['eW'], params['eb'],
                                        ceW_all, ceb_all, WE_stack)

    h = _mm(x, params['inW'], params['inb'], 1000)

    for li_, lp in enumerate(layers):
        Wab = jnp.concatenate([WA_all[li_], WB_all[li_]], axis=1)
        bab = jnp.concatenate([lp['preb'].reshape(TF),
                               jnp.zeros((TF,), jnp.float32)])
        AB = _mm(h, Wab, bab, 1000)
        A, B = AB[:, :TF], AB[:, TF:]

        C = _mm(ea_s, MC_all[li_], cC_all[li_, 0], 2000)

        d = B[src_s] + C
        S1 = jax.ops.segment_sum(d, dst_s, num_segments=N_NODES,
                                 indices_are_sorted=True)
        S2 = jax.ops.segment_sum(d * d, dst_s, num_segments=N_NODES,
                                 indices_are_sorted=True)
        MX = jax.ops.segment_max(d, dst_s, num_segments=N_NODES,
                                 indices_are_sorted=True)
        MN = jax.ops.segment_min(d, dst_s, num_segments=N_NODES,
                                 indices_are_sorted=True)
        MX = jnp.where(jnp.isfinite(MX), MX, 0.0)
        MN = jnp.where(jnp.isfinite(MN), MN, 0.0)

        pw = lp['postW']
        Whf = pw[:, :F, :].transpose(1, 0, 2).reshape(F, HID)
        Wstack = jnp.concatenate([pw[:, F:5 * F, :], pw[:, 5 * F:9 * F, :],
                                  pw[:, 9 * F:, :]], axis=2)
        postb_f = lp['postb'].reshape(HID)
        h = _post(A, S1, S2, MX, MN, h, cntf, Whf, Wstack, postb_f,
                  lp['linW'], lp['linb'], lp['bn_g'], lp['bn_b'],
                  lp['bn_rm'], lp['bn_rv'])

    addp, cntrep = _pool(batch, h)
    z = _mlp(addp, cntrep, params['clf'])
    return z[:, 0]


# speedup vs baseline: 17.5182x; 1.4468x over previous
import functools
import numpy as np
import jax
import jax.numpy as jnp
from jax import lax
from jax.experimental import pallas as pl
from jax.experimental.pallas import tpu as pltpu
from jax.experimental.pallas import tpu_sc as plsc

N_NODES = 10000
N_EDGES = 160000
HID = 128
T = 4
F = 128
FO = 32
TF = T * F
NGR = 64
AVG_LOG_C = float(np.log(np.arange(51, dtype=np.float64) + 1.0).sum() / 51.0)
EPS_BN = 1e-5


_NPW = 320
_GN = 8
_EC = 64
_NCH = TF // 16
_RPC = 336
_RP_PAD = 31 * _NPW + _RPC
_E_PAD = N_EDGES + 2000
_BIG = 3.0e38


def _rps(rp_v, j):
    return rp_v[pl.ds(j, 16)][0]


def _sc_body(m_hbm, perm_hbm, rp_hbm, out_hbm, rp_v, idx_v, mbuf, acc, sem):
    wid = lax.axis_index("s") * 2 + lax.axis_index("c")
    n0 = wid * _NPW
    pltpu.sync_copy(rp_hbm.at[pl.ds(n0, _RPC)], rp_v)
    nn = jnp.minimum(N_NODES - n0, _NPW)
    ngroups = nn // _GN

    def group_body(g, _g):
        gn0 = g * _GN
        e0 = _rps(rp_v, gn0)
        e1 = _rps(rp_v, gn0 + _GN)

        def init_body(r, _r):
            a = lax.rem(r, 4)
            val = jnp.where(a <= 1, 0.0, jnp.where(a == 2, -_BIG, _BIG))
            vec = jnp.full((16,), 1.0, jnp.float32) * val
            for c in range(_NCH):
                acc[r, pl.ds(c * 16, 16)] = vec
            return 0

        lax.fori_loop(0, _GN * 4, init_body, 0)

        ea8 = jnp.bitwise_and(e0, jnp.int32(-8))
        nch = lax.div(e1 - ea8 + (_EC - 1), jnp.int32(_EC))

        def chunk_body(k, _k):
            eb = pl.multiple_of(ea8 + k * _EC, 8)
            pltpu.sync_copy(perm_hbm.at[pl.ds(eb, _EC)], idx_v)
            pltpu.async_copy(m_hbm.at[idx_v], mbuf, sem).wait()

            def node_body(i, _i):
                lo = jnp.maximum(_rps(rp_v, gn0 + i), eb)
                hi = jnp.minimum(_rps(rp_v, gn0 + i + 1), eb + _EC)

                @pl.when(hi > lo)
                def _():
                    for cg in range(_NCH // 4):
                        regs = []
                        for a in range(4):
                            for j in range(4):
                                regs.append(acc[i * 4 + a,
                                                pl.ds((cg * 4 + j) * 16, 16)])

                        def e_body(e, cr):
                            el = e - eb
                            out = list(cr)
                            for j in range(4):
                                d = mbuf[el, pl.ds((cg * 4 + j) * 16, 16)]
                                out[j] = cr[j] + d
                                out[4 + j] = cr[4 + j] + d * d
                                out[8 + j] = jnp.maximum(cr[8 + j], d)
                                out[12 + j] = jnp.minimum(cr[12 + j], d)
                            return tuple(out)

                        res = lax.fori_loop(lo, hi, e_body, tuple(regs))
                        for a in range(4):
                            for j in range(4):
                                acc[i * 4 + a, pl.ds((cg * 4 + j) * 16, 16)] = (
                                    res[a * 4 + j])
                return 0

            lax.fori_loop(0, _GN, node_body, 0)
            return 0

        lax.fori_loop(0, nch, chunk_body, 0)
        pltpu.sync_copy(acc, out_hbm.at[pl.ds((n0 + gn0) * 4, _GN * 4)])
        return 0

    lax.fori_loop(0, ngroups, group_body, 0)


def _sc_segment(m2d, perm_pad, rp_pad):
    mesh = plsc.VectorSubcoreMesh(core_axis_name="c", subcore_axis_name="s")
    f = functools.partial(
        pl.kernel, mesh=mesh,
        out_type=jax.ShapeDtypeStruct((N_NODES * 4, TF), jnp.float32),
        scratch_types=[
            pltpu.VMEM((_RPC,), jnp.int32),
            pltpu.VMEM((_EC,), jnp.int32),
            pltpu.VMEM((_EC, TF), jnp.float32),
            pltpu.VMEM((_GN * 4, TF), jnp.float32),
            pltpu.SemaphoreType.DMA,
        ],
    )(_sc_body)
    return f(m2d, perm_pad, rp_pad)


def _bnx(z, g, b, rm, rv):
    return (z - rm) / jnp.sqrt(rv + EPS_BN) * g + b


def _conv(h, src, dst, ea, lp, perm_pad, rp_pad, cnt):
    N = h.shape[0]
    e = ea @ lp['ceW'] + lp['ceb']
    xi = h[dst]
    xj = h[src]
    m = (jnp.einsum('ei,tio->eto', xi, lp['preW'][:, :F, :])
         + jnp.einsum('ei,tio->eto', xj, lp['preW'][:, F:2 * F, :])
         + jnp.einsum('ei,tio->eto', e, lp['preW'][:, 2 * F:, :])
         + lp['preb'][None, :, :])

    agg4 = _sc_segment(m.reshape(N_EDGES, TF), perm_pad, rp_pad)
    agg4 = agg4.reshape(N_NODES, 4, T, F)
    s = agg4[:, 0]
    s2 = agg4[:, 1]
    mx = agg4[:, 2]
    mn = agg4[:, 3]

    cntc = jnp.clip(cnt, 1.0, None)
    mean = s / cntc[:, None, None]
    mean2 = s2 / cntc[:, None, None]
    var = jnp.maximum(mean2 - mean * mean, 0.0)
    std = jnp.sqrt(var + 1e-5)
    has = (cnt > 0)[:, None, None]
    mx = jnp.where(has, mx, 0.0)
    mn = jnp.where(has, mn, 0.0)
    agg = jnp.concatenate([mean, mx, mn, std], axis=-1)
    logdeg = jnp.log(cntc + 1.0)[:, None, None]
    out = jnp.concatenate([agg, agg * (logdeg / AVG_LOG_C),
                           agg * (AVG_LOG_C / logdeg)], axis=-1)
    ht = jnp.broadcast_to(h[:, None, :], (N, T, F))
    out = jnp.concatenate([ht, out], axis=-1)
    out = jnp.einsum('nti,tio->nto', out, lp['postW']) + lp['postb'][None]
    out = out.reshape(N, T * FO)
    return out @ lp['linW'] + lp['linb']


def kernel(x, edge_index, edge_attr, batch, params):
    src, dst = edge_index[0], edge_index[1]

    perm = jnp.argsort(dst, stable=True).astype(jnp.int32)
    dst_s = dst[perm]
    row_ptr = jnp.searchsorted(
        dst_s, jnp.arange(N_NODES + 1, dtype=jnp.int32)).astype(jnp.int32)
    cnt = (row_ptr[1:] - row_ptr[:-1]).astype(jnp.float32)
    rp_pad = jnp.pad(row_ptr, (0, _RP_PAD - (N_NODES + 1)),
                     constant_values=N_EDGES)
    perm_pad = jnp.concatenate([
        perm, (jnp.arange(_E_PAD - N_EDGES, dtype=jnp.int32) * 37) % N_EDGES])

    h = x @ params['inW'] + params['inb']
    ea = edge_attr @ params['eW'] + params['eb']
    for lp in params['layers']:
        hr = h
        h = _conv(h, src, dst, ea, lp, perm_pad, rp_pad, cnt)
        h = _bnx(h, lp['bn_g'], lp['bn_b'], lp['bn_rm'], lp['bn_rv'])
        h = jax.nn.relu(h)
        h = h + hr

    gcnt = jax.ops.segment_sum(jnp.ones((h.shape[0],), jnp.float32), batch,
                               num_segments=NGR)
    addp = jax.ops.segment_sum(h, batch, num_segments=NGR)
    meanp = addp / jnp.clip(gcnt, 1.0, None)[:, None]
    g = jnp.concatenate([meanp, addp], axis=-1)
    c = params['clf']
    z = g @ c['W1'] + c['b1']
    z = _bnx(z, c['bn_g'], c['bn_b'], c['bn_rm'], c['bn_rv'])
    z = jax.nn.relu(z)
    z = jax.nn.relu(z @ c['W2'] + c['b2'])
    z = z @ c['W3'] + c['b3']
    return z[:, 0]
